# baseline (speedup 1.0000x reference)
import jax
import jax.numpy as jnp
from jax import lax
from jax.experimental import pallas as pl
from jax.experimental.pallas import tpu as pltpu

INPUT = 1024
HIDDEN = 4096
OUTPUT = 1024
E = 8
N = 2048
ICHUNK = 256
NI = INPUT // ICHUNK
HCHUNK = 1024
NC = HIDDEN // HCHUNK
SPE = NI + NC
T = E * SPE

_SQRT_HALF = 0.7071067811865476


def _fused_body(x_ref, wr_ref, br_ref, w1_ref, b1_ref, w2_ref, b2_ref,
                gamma_ref, beta_ref, out_ref,
                wcomb_s, xfirst_s, h_s, g_s, f_s):
    t = pl.program_id(0)
    e = t // SPE
    s = t % SPE

    @pl.when(t == 0)
    def _router():
        x = x_ref[...]
        logits = jnp.dot(x, wr_ref[...], preferred_element_type=jnp.float32)
        logits = logits + br_ref[...]

        iota_e = lax.broadcasted_iota(jnp.int32, (N, E), 1)
        m1 = jnp.max(logits, axis=-1, keepdims=True)
        a1 = jnp.min(jnp.where(logits == m1, iota_e, E), axis=-1, keepdims=True)
        masked = jnp.where(iota_e == a1, -jnp.inf, logits)
        m2 = jnp.max(masked, axis=-1, keepdims=True)
        a2 = jnp.min(jnp.where(masked == m2, iota_e, E), axis=-1, keepdims=True)

        r = jnp.exp(m2 - m1)
        denom = 1.0 + r
        p1 = 1.0 / denom
        p2 = r / denom

        sel1 = iota_e == a1
        sel2 = iota_e == a2
        wcomb_s[...] = jnp.where(sel1, p1, 0.0) + jnp.where(sel2, p2, 0.0)

        sel = sel1 | sel2
        iota_n = lax.broadcasted_iota(jnp.int32, (N, E), 0)
        fi = jnp.min(jnp.where(sel, iota_n, N), axis=0, keepdims=True)
        onehot = (iota_n == fi).astype(jnp.float32)
        xfirst = lax.dot_general(
            onehot, x, (((0,), (0,)), ((), ())),
            preferred_element_type=jnp.float32)
        for i in range(NI):
            xfirst_s[i] = xfirst[:, i * ICHUNK:(i + 1) * ICHUNK]
        f_s[...] = b2_ref[...]

    iota_row = lax.broadcasted_iota(jnp.int32, (1, E), 1)
    oh_e = (iota_row == e).astype(jnp.float32)

    @pl.when(s < NI)
    def _w1_phase():
        xr_chunk = jnp.dot(oh_e, xfirst_s[s],
                           preferred_element_type=jnp.float32)
        part = jnp.dot(xr_chunk, w1_ref[0],
                       preferred_element_type=jnp.float32)

        @pl.when(s == 0)
        def _():
            b1row = jnp.dot(oh_e, b1_ref[...],
                            preferred_element_type=jnp.float32)
            h_s[...] = part + b1row

        @pl.when(s != 0)
        def _():
            h_s[...] += part

        @pl.when(s == NI - 1)
        def _gelu():
            h = h_s[...]
            g = 0.5 * h * (1.0 + lax.erf(h * _SQRT_HALF))
            for j in range(NC):
                g_s[j] = g[:, j * HCHUNK:(j + 1) * HCHUNK]

    @pl.when(s >= NI)
    def _w2_phase():
        c = s - NI
        part = jnp.dot(g_s[c], w2_ref[0],
                       preferred_element_type=jnp.float32)
        rmask = (lax.broadcasted_iota(jnp.int32, (E, 1), 0) == e)
        f_s[...] += rmask.astype(jnp.float32) * part

    @pl.when(t == T - 1)
    def _combine():
        pre = jnp.dot(wcomb_s[...], f_s[...],
                      preferred_element_type=jnp.float32)
        mean = jnp.mean(pre, axis=-1, keepdims=True)
        d = pre - mean
        var = jnp.mean(d * d, axis=-1, keepdims=True)
        inv = lax.rsqrt(var + 1e-5)
        out_ref[...] = d * inv * gamma_ref[...] + beta_ref[...]


@jax.jit
def kernel(x, Wr, br, W1, b1, W2, b2, gamma, beta):
    Bc, S, D = x.shape
    xf = x.reshape(Bc * S, D)

    def w1_map(t):
        e = t // SPE
        s = t % SPE
        return (e, jnp.minimum(s, NI - 1), 0)

    def w2_map(t):
        e = t // SPE
        s = t % SPE
        return (e, jnp.maximum(s - NI, 0), 0)

    out = pl.pallas_call(
        _fused_body,
        grid=(T,),
        in_specs=[
            pl.BlockSpec((N, INPUT), lambda t: (0, 0)),
            pl.BlockSpec((INPUT, E), lambda t: (0, 0)),
            pl.BlockSpec((1, E), lambda t: (0, 0)),
            pl.BlockSpec((1, ICHUNK, HIDDEN), w1_map),
            pl.BlockSpec((E, HIDDEN), lambda t: (0, 0)),
            pl.BlockSpec((1, HCHUNK, OUTPUT), w2_map),
            pl.BlockSpec((E, OUTPUT), lambda t: (0, 0)),
            pl.BlockSpec((1, OUTPUT), lambda t: (0, 0)),
            pl.BlockSpec((1, OUTPUT), lambda t: (0, 0)),
        ],
        out_specs=pl.BlockSpec((N, OUTPUT), lambda t: (0, 0)),
        out_shape=jax.ShapeDtypeStruct((N, OUTPUT), jnp.float32),
        scratch_shapes=[
            pltpu.VMEM((N, E), jnp.float32),
            pltpu.VMEM((NI, E, ICHUNK), jnp.float32),
            pltpu.VMEM((1, HIDDEN), jnp.float32),
            pltpu.VMEM((NC, 1, HCHUNK), jnp.float32),
            pltpu.VMEM((E, OUTPUT), jnp.float32),
        ],
    )(xf, Wr, br.reshape(1, E), W1, b1, W2, b2,
      gamma.reshape(1, OUTPUT), beta.reshape(1, OUTPUT))

    return out.reshape(Bc, S, OUTPUT)


# speedup vs baseline: 1.1721x; 1.1721x over previous
import jax
import jax.numpy as jnp
from jax import lax
from jax.experimental import pallas as pl
from jax.experimental.pallas import tpu as pltpu

INPUT = 1024
HIDDEN = 4096
OUTPUT = 1024
E = 8
N = 2048
HCHUNK = 1024
NCH = HIDDEN // HCHUNK
NP = NCH // 2
T = E * NP

_SQRT_HALF = 0.7071067811865476


def _fused_body(x_ref, wr_ref, br_ref, w1a_ref, w1b_ref, b1a_ref, b1b_ref,
                w2a_ref, w2b_ref, b2_ref,
                gamma_ref, beta_ref, out_ref, wcomb_s, xfirst_s, f_s):
    t = pl.program_id(0)
    e = t // NP

    @pl.when(t == 0)
    def _router():
        x = x_ref[...]
        logits = jnp.dot(x, wr_ref[...], preferred_element_type=jnp.float32)
        logits = logits + br_ref[...]

        iota_e = lax.broadcasted_iota(jnp.int32, (N, E), 1)
        m1 = jnp.max(logits, axis=-1, keepdims=True)
        a1 = jnp.min(jnp.where(logits == m1, iota_e, E), axis=-1, keepdims=True)
        masked = jnp.where(iota_e == a1, -jnp.inf, logits)
        m2 = jnp.max(masked, axis=-1, keepdims=True)
        a2 = jnp.min(jnp.where(masked == m2, iota_e, E), axis=-1, keepdims=True)

        r = jnp.exp(m2 - m1)
        denom = 1.0 + r
        p1 = 1.0 / denom
        p2 = r / denom

        sel1 = iota_e == a1
        sel2 = iota_e == a2
        wcomb_s[...] = jnp.where(sel1, p1, 0.0) + jnp.where(sel2, p2, 0.0)

        sel = sel1 | sel2
        iota_n = lax.broadcasted_iota(jnp.int32, (N, E), 0)
        fi = jnp.min(jnp.where(sel, iota_n, N), axis=0, keepdims=True)
        onehot = (iota_n == fi).astype(jnp.float32)
        xfirst_s[...] = lax.dot_general(
            onehot, x, (((0,), (0,)), ((), ())),
            preferred_element_type=jnp.float32)
        f_s[...] = b2_ref[:, 0, :]

    iota_row = lax.broadcasted_iota(jnp.int32, (1, E), 1)
    oh_e = (iota_row == e).astype(jnp.float32)
    xr = jnp.dot(oh_e, xfirst_s[...], preferred_element_type=jnp.float32)

    part = None
    for w1_ref, b1_ref, w2_ref in ((w1a_ref, b1a_ref, w2a_ref),
                                   (w1b_ref, b1b_ref, w2b_ref)):
        h = jnp.dot(xr, w1_ref[0], preferred_element_type=jnp.float32)
        h = h + b1_ref[0]
        g = 0.5 * h * (1.0 + lax.erf(h * _SQRT_HALF))
        p = jnp.dot(g, w2_ref[0], preferred_element_type=jnp.float32)
        part = p if part is None else part + p

    rmask = (lax.broadcasted_iota(jnp.int32, (E, 1), 0) == e).astype(jnp.float32)
    f_s[...] += rmask * part

    @pl.when(t == T - 1)
    def _combine():
        pre = jnp.dot(wcomb_s[...], f_s[...],
                      preferred_element_type=jnp.float32)
        mean = jnp.mean(pre, axis=-1, keepdims=True)
        d = pre - mean
        var = jnp.mean(d * d, axis=-1, keepdims=True)
        inv = lax.rsqrt(var + 1e-5)
        out_ref[...] = d * inv * gamma_ref[...] + beta_ref[...]


@jax.jit
def kernel(x, Wr, br, W1, b1, W2, b2, gamma, beta):
    Bc, S, D = x.shape
    xf = x.reshape(Bc * S, D)

    out = pl.pallas_call(
        _fused_body,
        grid=(T,),
        in_specs=[
            pl.BlockSpec((N, INPUT), lambda t: (0, 0)),
            pl.BlockSpec((INPUT, E), lambda t: (0, 0)),
            pl.BlockSpec((1, E), lambda t: (0, 0)),
            pl.BlockSpec((1, INPUT, HCHUNK),
                         lambda t: (t // NP, 0, 2 * (t % NP))),
            pl.BlockSpec((1, INPUT, HCHUNK),
                         lambda t: (t // NP, 0, 2 * (t % NP) + 1)),
            pl.BlockSpec((1, 1, HCHUNK),
                         lambda t: (NCH * (t // NP) + 2 * (t % NP), 0, 0)),
            pl.BlockSpec((1, 1, HCHUNK),
                         lambda t: (NCH * (t // NP) + 2 * (t % NP) + 1, 0, 0)),
            pl.BlockSpec((1, HCHUNK, OUTPUT),
                         lambda t: (t // NP, 2 * (t % NP), 0)),
            pl.BlockSpec((1, HCHUNK, OUTPUT),
                         lambda t: (t // NP, 2 * (t % NP) + 1, 0)),
            pl.BlockSpec((E, 1, OUTPUT), lambda t: (0, 0, 0)),
            pl.BlockSpec((1, OUTPUT), lambda t: (0, 0)),
            pl.BlockSpec((1, OUTPUT), lambda t: (0, 0)),
        ],
        out_specs=pl.BlockSpec((N, OUTPUT), lambda t: (0, 0)),
        out_shape=jax.ShapeDtypeStruct((N, OUTPUT), jnp.float32),
        scratch_shapes=[
            pltpu.VMEM((N, E), jnp.float32),
            pltpu.VMEM((E, INPUT), jnp.float32),
            pltpu.VMEM((E, OUTPUT), jnp.float32),
        ],
    )(xf, Wr, br.reshape(1, E), W1, W1,
      b1.reshape(E * NCH, 1, HCHUNK), b1.reshape(E * NCH, 1, HCHUNK),
      W2, W2, b2.reshape(E, 1, OUTPUT), gamma.reshape(1, OUTPUT),
      beta.reshape(1, OUTPUT))

    return out.reshape(Bc, S, OUTPUT)
